# baseline (speedup 1.0000x reference)
import functools

import jax
import jax.numpy as jnp
from jax import lax
from jax.experimental import pallas as pl
from jax.experimental.pallas import tpu as pltpu
from jax.experimental.pallas import tpu_sc as plsc

N, E, D, H, C, B = 10000, 320000, 128, 128, 64, 16

NC, NS = 2, 16
NW = NC * NS
FPW = D // NW
ECH = 4000
NCHUNKS = E // ECH
NB = 2
LANES = 16

def _edge_agg_body(xT_hbm, src_hbm, dst_hbm, out_hbm, xv, acc,
                   srcv0, dstv0, srcv1, dstv1, ssem0, dsem0, ssem1, dsem1):
    wid = lax.axis_index("s") * NC + lax.axis_index("c")
    base = wid * FPW
    srcs, dsts = (srcv0, srcv1), (dstv0, dstv1)
    ssems, dsems = (ssem0, ssem1), (dsem0, dsem1)

    def _start(b, ci):
        e0 = ci * ECH
        pltpu.make_async_copy(src_hbm.at[pl.ds(e0, ECH)], srcs[b],
                              ssems[b]).start()
        pltpu.make_async_copy(dst_hbm.at[pl.ds(e0, ECH)], dsts[b],
                              dsems[b]).start()

    def _wait(b):
        pltpu.make_async_copy(src_hbm.at[pl.ds(0, ECH)], srcs[b],
                              ssems[b]).wait()
        pltpu.make_async_copy(dst_hbm.at[pl.ds(0, ECH)], dsts[b],
                              dsems[b]).wait()

    for b in range(NB):
        _start(b, b)

    pltpu.sync_copy(xT_hbm.at[pl.ds(base * N, FPW * N)], xv)

    zero16 = jnp.zeros((LANES,), jnp.float32)

    def _zero(i, _):
        acc[pl.ds(i * LANES, LANES)] = zero16
        return 0

    lax.fori_loop(0, FPW * N // LANES, _zero, 0, unroll=8)

    def _pair(p, _):
        for b in range(NB):
            ci = p * NB + b
            _wait(b)
            sv, dv = srcs[b], dsts[b]

            @plsc.parallel_loop(0, ECH // LANES, unroll=16)
            def _edges(j):
                s16 = sv[pl.ds(j * LANES, LANES)]
                d16 = dv[pl.ds(j * LANES, LANES)]
                for f in range(FPW):
                    v = plsc.load_gather(xv, [s16 + (f * N)])
                    plsc.addupdate_scatter(acc, [d16 + (f * N)], v)

            @pl.when(ci + NB < NCHUNKS)
            def _():
                _start(b, ci + NB)
        return 0

    lax.fori_loop(0, NCHUNKS // NB, _pair, 0)
    pltpu.sync_copy(acc, out_hbm.at[pl.ds(base * N, FPW * N)])


@functools.lru_cache(maxsize=1)
def _edge_agg():
    mesh = plsc.VectorSubcoreMesh(core_axis_name="c", subcore_axis_name="s",
                                  num_cores=NC, num_subcores=NS)
    return pl.kernel(
        _edge_agg_body,
        out_type=jax.ShapeDtypeStruct((D * N,), jnp.float32),
        mesh=mesh,
        compiler_params=pltpu.CompilerParams(needs_layout_passes=False),
        scratch_types=[
            pltpu.VMEM((FPW * N,), jnp.float32),
            pltpu.VMEM((FPW * N,), jnp.float32),
            pltpu.VMEM((ECH,), jnp.int32),
            pltpu.VMEM((ECH,), jnp.int32),
            pltpu.VMEM((ECH,), jnp.int32),
            pltpu.VMEM((ECH,), jnp.int32),
            pltpu.SemaphoreType.DMA,
            pltpu.SemaphoreType.DMA,
            pltpu.SemaphoreType.DMA,
            pltpu.SemaphoreType.DMA,
        ],
    )


NEG = float("-inf")


def _probs_body(x_ref, W1_ref, b1_ref, W2_ref, b2_ref, W_root_ref,
                b_rel_ref, probs_ref, xroot_ref):
    x = x_ref[...]

    h = jnp.maximum(
        jnp.dot(x, W1_ref[...], preferred_element_type=jnp.float32)
        + b1_ref[...], 0.0)
    logits = (jnp.dot(h, W2_ref[...], preferred_element_type=jnp.float32)
              + b2_ref[...])
    m = jnp.max(logits, axis=-1, keepdims=True)
    e = jnp.exp(logits - m)
    probs_ref[...] = e / jnp.sum(e, axis=-1, keepdims=True)

    xroot_ref[...] = (
        jnp.dot(x, W_root_ref[...], preferred_element_type=jnp.float32)
        + b_rel_ref[...])


_probs = pl.pallas_call(
    _probs_body,
    out_shape=[
        jax.ShapeDtypeStruct((N, C), jnp.float32),
        jax.ShapeDtypeStruct((N, H), jnp.float32),
    ],
)


def _dense_body(aggT_ref, xroot_ref, batch_ref, batch_col_ref,
                W_rel_ref, W_out_ref, b_out_ref, ge_ref):
    aggT = aggT_ref[...]
    conv = (lax.dot_general(aggT, W_rel_ref[...],
                            (((0,), (0,)), ((), ())),
                            preferred_element_type=jnp.float32)
            + xroot_ref[...])

    bt_row = batch_ref[...]
    bt_col = batch_col_ref[...]
    seg = lax.broadcasted_iota(jnp.int32, (B, N), 0)
    onehot = (bt_row == seg).astype(jnp.float32)
    addp = jnp.dot(onehot, conv, preferred_element_type=jnp.float32)
    cnt = jnp.broadcast_to(jnp.sum(onehot, axis=1, keepdims=True), (B, H))
    meanp = addp / jnp.maximum(cnt, 1.0)

    rows = []
    for b in range(B):
        mask = bt_col == b
        rows.append(jnp.max(jnp.where(mask, conv, NEG), axis=0,
                            keepdims=True))
    maxp = jnp.concatenate(rows, axis=0)

    W_out = W_out_ref[...]
    ge_ref[...] = (
        jnp.dot(meanp, W_out[0:H, :], preferred_element_type=jnp.float32)
        + jnp.dot(maxp, W_out[H:2 * H, :], preferred_element_type=jnp.float32)
        + jnp.dot(addp, W_out[2 * H:3 * H, :],
                  preferred_element_type=jnp.float32)
        + b_out_ref[...])


_dense = pl.pallas_call(
    _dense_body,
    out_shape=jax.ShapeDtypeStruct((B, H), jnp.float32),
)


def kernel(x, edge_index, batch, W1, b1, W2, b2, W_rel, b_rel, W_root,
           W_out, b_out):
    xT = x.T.reshape(-1)
    aggT = _edge_agg()(xT, edge_index[0], edge_index[1]).reshape(D, N)
    probs, xroot = _probs(
        x, W1, b1.reshape(1, H), W2, b2.reshape(1, C), W_root,
        b_rel.reshape(1, H))
    ge = _dense(
        aggT, xroot, batch.reshape(1, N), batch.reshape(N, 1),
        W_rel, W_out, b_out.reshape(1, H))
    return (ge, probs)


# speedup vs baseline: 1.1402x; 1.1402x over previous
import functools

import jax
import jax.numpy as jnp
from jax import lax
from jax.experimental import pallas as pl
from jax.experimental.pallas import tpu as pltpu
from jax.experimental.pallas import tpu_sc as plsc

N, E, D, H, C, B = 10000, 320000, 128, 128, 64, 16

NC, NS = 2, 16
NW = NC * NS
FPW = D // NW
ECH = 4000
NCHUNKS = E // ECH
NB = 2
LANES = 16

def _edge_agg_body(xT_hbm, src_hbm, dst_hbm, out_hbm, xv, acc,
                   srcv0, dstv0, srcv1, dstv1, ssem0, dsem0, ssem1, dsem1):
    wid = lax.axis_index("s") * NC + lax.axis_index("c")
    base = wid * FPW
    srcs, dsts = (srcv0, srcv1), (dstv0, dstv1)
    ssems, dsems = (ssem0, ssem1), (dsem0, dsem1)

    def _start(b, ci):
        e0 = ci * ECH
        pltpu.make_async_copy(src_hbm.at[pl.ds(e0, ECH)], srcs[b],
                              ssems[b]).start()
        pltpu.make_async_copy(dst_hbm.at[pl.ds(e0, ECH)], dsts[b],
                              dsems[b]).start()

    def _wait(b):
        pltpu.make_async_copy(src_hbm.at[pl.ds(0, ECH)], srcs[b],
                              ssems[b]).wait()
        pltpu.make_async_copy(dst_hbm.at[pl.ds(0, ECH)], dsts[b],
                              dsems[b]).wait()

    for b in range(NB):
        _start(b, b)

    pltpu.sync_copy(xT_hbm.at[pl.ds(base * N, FPW * N)], xv)

    zero16 = jnp.zeros((LANES,), jnp.float32)

    def _zero(i, _):
        acc[pl.ds(i * LANES, LANES)] = zero16
        return 0

    lax.fori_loop(0, FPW * N // LANES, _zero, 0, unroll=8)

    def _pair(p, _):
        for b in range(NB):
            ci = p * NB + b
            _wait(b)
            sv, dv = srcs[b], dsts[b]

            @plsc.parallel_loop(0, ECH // LANES, unroll=4)
            def _edges(j):
                s16 = sv[pl.ds(j * LANES, LANES)]
                d16 = dv[pl.ds(j * LANES, LANES)]
                for f in range(FPW):
                    v = plsc.load_gather(xv, [s16 + (f * N)])
                    plsc.addupdate_scatter(acc, [d16 + (f * N)], v)

            @pl.when(ci + NB < NCHUNKS)
            def _():
                _start(b, ci + NB)
        return 0

    lax.fori_loop(0, NCHUNKS // NB, _pair, 0)
    pltpu.sync_copy(acc, out_hbm.at[pl.ds(base * N, FPW * N)])


@functools.lru_cache(maxsize=1)
def _edge_agg():
    mesh = plsc.VectorSubcoreMesh(core_axis_name="c", subcore_axis_name="s",
                                  num_cores=NC, num_subcores=NS)
    return pl.kernel(
        _edge_agg_body,
        out_type=jax.ShapeDtypeStruct((D * N,), jnp.float32),
        mesh=mesh,
        compiler_params=pltpu.CompilerParams(needs_layout_passes=False),
        scratch_types=[
            pltpu.VMEM((FPW * N,), jnp.float32),
            pltpu.VMEM((FPW * N,), jnp.float32),
            pltpu.VMEM((ECH,), jnp.int32),
            pltpu.VMEM((ECH,), jnp.int32),
            pltpu.VMEM((ECH,), jnp.int32),
            pltpu.VMEM((ECH,), jnp.int32),
            pltpu.SemaphoreType.DMA,
            pltpu.SemaphoreType.DMA,
            pltpu.SemaphoreType.DMA,
            pltpu.SemaphoreType.DMA,
        ],
    )


NEG = float("-inf")


def _probs_body(x_ref, W1_ref, b1_ref, W2_ref, b2_ref, W_root_ref,
                b_rel_ref, probs_ref, xroot_ref):
    x = x_ref[...]

    h = jnp.maximum(
        jnp.dot(x, W1_ref[...], preferred_element_type=jnp.float32)
        + b1_ref[...], 0.0)
    logits = (jnp.dot(h, W2_ref[...], preferred_element_type=jnp.float32)
              + b2_ref[...])
    m = jnp.max(logits, axis=-1, keepdims=True)
    e = jnp.exp(logits - m)
    probs_ref[...] = e / jnp.sum(e, axis=-1, keepdims=True)

    xroot_ref[...] = (
        jnp.dot(x, W_root_ref[...], preferred_element_type=jnp.float32)
        + b_rel_ref[...])


_probs = pl.pallas_call(
    _probs_body,
    out_shape=[
        jax.ShapeDtypeStruct((N, C), jnp.float32),
        jax.ShapeDtypeStruct((N, H), jnp.float32),
    ],
)


def _dense_body(aggT_ref, xroot_ref, batch_ref, batch_col_ref,
                W_rel_ref, W_out_ref, b_out_ref, ge_ref):
    aggT = aggT_ref[...]
    conv = (lax.dot_general(aggT, W_rel_ref[...],
                            (((0,), (0,)), ((), ())),
                            preferred_element_type=jnp.float32)
            + xroot_ref[...])

    bt_row = batch_ref[...]
    bt_col = batch_col_ref[...]
    seg = lax.broadcasted_iota(jnp.int32, (B, N), 0)
    onehot = (bt_row == seg).astype(jnp.float32)
    addp = jnp.dot(onehot, conv, preferred_element_type=jnp.float32)
    cnt = jnp.broadcast_to(jnp.sum(onehot, axis=1, keepdims=True), (B, H))
    meanp = addp / jnp.maximum(cnt, 1.0)

    rows = []
    for b in range(B):
        mask = bt_col == b
        rows.append(jnp.max(jnp.where(mask, conv, NEG), axis=0,
                            keepdims=True))
    maxp = jnp.concatenate(rows, axis=0)

    W_out = W_out_ref[...]
    ge_ref[...] = (
        jnp.dot(meanp, W_out[0:H, :], preferred_element_type=jnp.float32)
        + jnp.dot(maxp, W_out[H:2 * H, :], preferred_element_type=jnp.float32)
        + jnp.dot(addp, W_out[2 * H:3 * H, :],
                  preferred_element_type=jnp.float32)
        + b_out_ref[...])


_dense = pl.pallas_call(
    _dense_body,
    out_shape=jax.ShapeDtypeStruct((B, H), jnp.float32),
)


def kernel(x, edge_index, batch, W1, b1, W2, b2, W_rel, b_rel, W_root,
           W_out, b_out):
    xT = x.T.reshape(-1)
    aggT = _edge_agg()(xT, edge_index[0], edge_index[1]).reshape(D, N)
    probs, xroot = _probs(
        x, W1, b1.reshape(1, H), W2, b2.reshape(1, C), W_root,
        b_rel.reshape(1, H))
    ge = _dense(
        aggT, xroot, batch.reshape(1, N), batch.reshape(N, 1),
        W_rel, W_out, b_out.reshape(1, H))
    return (ge, probs)


# speedup vs baseline: 1.1671x; 1.0236x over previous
import functools

import jax
import jax.numpy as jnp
from jax import lax
from jax.experimental import pallas as pl
from jax.experimental.pallas import tpu as pltpu
from jax.experimental.pallas import tpu_sc as plsc

N, E, D, H, C, B = 10000, 320000, 128, 128, 64, 16

NC, NS = 2, 16
NW = NC * NS
FPW = D // NW
ECH = 4000
NCHUNKS = E // ECH
NB = 2
LANES = 16

def _edge_agg_body(xT_hbm, src_hbm, dst_hbm, out_hbm, xv, acc,
                   srcv0, dstv0, srcv1, dstv1, ssem0, dsem0, ssem1, dsem1):
    wid = lax.axis_index("s") * NC + lax.axis_index("c")
    base = wid * FPW
    srcs, dsts = (srcv0, srcv1), (dstv0, dstv1)
    ssems, dsems = (ssem0, ssem1), (dsem0, dsem1)

    def _start(b, ci):
        e0 = ci * ECH
        pltpu.make_async_copy(src_hbm.at[pl.ds(e0, ECH)], srcs[b],
                              ssems[b]).start()
        pltpu.make_async_copy(dst_hbm.at[pl.ds(e0, ECH)], dsts[b],
                              dsems[b]).start()

    def _wait(b):
        pltpu.make_async_copy(src_hbm.at[pl.ds(0, ECH)], srcs[b],
                              ssems[b]).wait()
        pltpu.make_async_copy(dst_hbm.at[pl.ds(0, ECH)], dsts[b],
                              dsems[b]).wait()

    for b in range(NB):
        _start(b, b)

    pltpu.sync_copy(xT_hbm.at[pl.ds(base * N, FPW * N)], xv)

    zero16 = jnp.zeros((LANES,), jnp.float32)

    def _zero(i, _):
        acc[pl.ds(i * LANES, LANES)] = zero16
        return 0

    lax.fori_loop(0, FPW * N // LANES, _zero, 0, unroll=8)

    def _pair(p, _):
        for b in range(NB):
            ci = p * NB + b
            _wait(b)
            sv, dv = srcs[b], dsts[b]

            @plsc.parallel_loop(0, ECH // LANES, unroll=2)
            def _edges(j):
                s16 = sv[pl.ds(j * LANES, LANES)]
                d16 = dv[pl.ds(j * LANES, LANES)]
                for f in range(FPW):
                    v = plsc.load_gather(xv, [s16 + (f * N)])
                    plsc.addupdate_scatter(acc, [d16 + (f * N)], v)

            @pl.when(ci + NB < NCHUNKS)
            def _():
                _start(b, ci + NB)
        return 0

    lax.fori_loop(0, NCHUNKS // NB, _pair, 0)
    pltpu.sync_copy(acc, out_hbm.at[pl.ds(base * N, FPW * N)])


@functools.lru_cache(maxsize=1)
def _edge_agg():
    mesh = plsc.VectorSubcoreMesh(core_axis_name="c", subcore_axis_name="s",
                                  num_cores=NC, num_subcores=NS)
    return pl.kernel(
        _edge_agg_body,
        out_type=jax.ShapeDtypeStruct((D * N,), jnp.float32),
        mesh=mesh,
        compiler_params=pltpu.CompilerParams(needs_layout_passes=False),
        scratch_types=[
            pltpu.VMEM((FPW * N,), jnp.float32),
            pltpu.VMEM((FPW * N,), jnp.float32),
            pltpu.VMEM((ECH,), jnp.int32),
            pltpu.VMEM((ECH,), jnp.int32),
            pltpu.VMEM((ECH,), jnp.int32),
            pltpu.VMEM((ECH,), jnp.int32),
            pltpu.SemaphoreType.DMA,
            pltpu.SemaphoreType.DMA,
            pltpu.SemaphoreType.DMA,
            pltpu.SemaphoreType.DMA,
        ],
    )


NEG = float("-inf")


def _probs_body(x_ref, W1_ref, b1_ref, W2_ref, b2_ref, W_root_ref,
                b_rel_ref, probs_ref, xroot_ref):
    x = x_ref[...]

    h = jnp.maximum(
        jnp.dot(x, W1_ref[...], preferred_element_type=jnp.float32)
        + b1_ref[...], 0.0)
    logits = (jnp.dot(h, W2_ref[...], preferred_element_type=jnp.float32)
              + b2_ref[...])
    m = jnp.max(logits, axis=-1, keepdims=True)
    e = jnp.exp(logits - m)
    probs_ref[...] = e / jnp.sum(e, axis=-1, keepdims=True)

    xroot_ref[...] = (
        jnp.dot(x, W_root_ref[...], preferred_element_type=jnp.float32)
        + b_rel_ref[...])


_probs = pl.pallas_call(
    _probs_body,
    out_shape=[
        jax.ShapeDtypeStruct((N, C), jnp.float32),
        jax.ShapeDtypeStruct((N, H), jnp.float32),
    ],
)


def _dense_body(aggT_ref, xroot_ref, batch_ref, batch_col_ref,
                W_rel_ref, W_out_ref, b_out_ref, ge_ref):
    aggT = aggT_ref[...]
    conv = (lax.dot_general(aggT, W_rel_ref[...],
                            (((0,), (0,)), ((), ())),
                            preferred_element_type=jnp.float32)
            + xroot_ref[...])

    bt_row = batch_ref[...]
    bt_col = batch_col_ref[...]
    seg = lax.broadcasted_iota(jnp.int32, (B, N), 0)
    onehot = (bt_row == seg).astype(jnp.float32)
    addp = jnp.dot(onehot, conv, preferred_element_type=jnp.float32)
    cnt = jnp.broadcast_to(jnp.sum(onehot, axis=1, keepdims=True), (B, H))
    meanp = addp / jnp.maximum(cnt, 1.0)

    rows = []
    for b in range(B):
        mask = bt_col == b
        rows.append(jnp.max(jnp.where(mask, conv, NEG), axis=0,
                            keepdims=True))
    maxp = jnp.concatenate(rows, axis=0)

    W_out = W_out_ref[...]
    ge_ref[...] = (
        jnp.dot(meanp, W_out[0:H, :], preferred_element_type=jnp.float32)
        + jnp.dot(maxp, W_out[H:2 * H, :], preferred_element_type=jnp.float32)
        + jnp.dot(addp, W_out[2 * H:3 * H, :],
                  preferred_element_type=jnp.float32)
        + b_out_ref[...])


_dense = pl.pallas_call(
    _dense_body,
    out_shape=jax.ShapeDtypeStruct((B, H), jnp.float32),
)


def kernel(x, edge_index, batch, W1, b1, W2, b2, W_rel, b_rel, W_root,
           W_out, b_out):
    xT = x.T.reshape(-1)
    aggT = _edge_agg()(xT, edge_index[0], edge_index[1]).reshape(D, N)
    probs, xroot = _probs(
        x, W1, b1.reshape(1, H), W2, b2.reshape(1, C), W_root,
        b_rel.reshape(1, H))
    ge = _dense(
        aggT, xroot, batch.reshape(1, N), batch.reshape(N, 1),
        W_rel, W_out, b_out.reshape(1, H))
    return (ge, probs)


# speedup vs baseline: 1.1728x; 1.0049x over previous
import functools

import jax
import jax.numpy as jnp
from jax import lax
from jax.experimental import pallas as pl
from jax.experimental.pallas import tpu as pltpu
from jax.experimental.pallas import tpu_sc as plsc

N, E, D, H, C, B = 10000, 320000, 128, 128, 64, 16

NC, NS = 2, 16
NW = NC * NS
FPW = D // NW
ECH = 4000
NCHUNKS = E // ECH
NB = 2
LANES = 16

def _edge_agg_body(xT_hbm, src_hbm, dst_hbm, out_hbm, xv, acc,
                   srcv0, dstv0, srcv1, dstv1, ssem0, dsem0, ssem1, dsem1):
    wid = lax.axis_index("s") * NC + lax.axis_index("c")
    base = wid * FPW
    srcs, dsts = (srcv0, srcv1), (dstv0, dstv1)
    ssems, dsems = (ssem0, ssem1), (dsem0, dsem1)

    def _start(b, ci):
        e0 = ci * ECH
        pltpu.make_async_copy(src_hbm.at[pl.ds(e0, ECH)], srcs[b],
                              ssems[b]).start()
        pltpu.make_async_copy(dst_hbm.at[pl.ds(e0, ECH)], dsts[b],
                              dsems[b]).start()

    def _wait(b):
        pltpu.make_async_copy(src_hbm.at[pl.ds(0, ECH)], srcs[b],
                              ssems[b]).wait()
        pltpu.make_async_copy(dst_hbm.at[pl.ds(0, ECH)], dsts[b],
                              dsems[b]).wait()

    for b in range(NB):
        _start(b, b)

    pltpu.sync_copy(xT_hbm.at[pl.ds(base * N, FPW * N)], xv)

    zero16 = jnp.zeros((LANES,), jnp.float32)

    def _zero(i, _):
        acc[pl.ds(i * LANES, LANES)] = zero16
        return 0

    lax.fori_loop(0, FPW * N // LANES, _zero, 0, unroll=8)

    def _pair(p, _):
        for b in range(NB):
            ci = p * NB + b
            _wait(b)
            sv, dv = srcs[b], dsts[b]

            @plsc.parallel_loop(0, ECH // LANES, unroll=1)
            def _edges(j):
                s16 = sv[pl.ds(j * LANES, LANES)]
                d16 = dv[pl.ds(j * LANES, LANES)]
                for f in range(FPW):
                    v = plsc.load_gather(xv, [s16 + (f * N)])
                    plsc.addupdate_scatter(acc, [d16 + (f * N)], v)

            @pl.when(ci + NB < NCHUNKS)
            def _():
                _start(b, ci + NB)
        return 0

    lax.fori_loop(0, NCHUNKS // NB, _pair, 0)
    pltpu.sync_copy(acc, out_hbm.at[pl.ds(base * N, FPW * N)])


@functools.lru_cache(maxsize=1)
def _edge_agg():
    mesh = plsc.VectorSubcoreMesh(core_axis_name="c", subcore_axis_name="s",
                                  num_cores=NC, num_subcores=NS)
    return pl.kernel(
        _edge_agg_body,
        out_type=jax.ShapeDtypeStruct((D * N,), jnp.float32),
        mesh=mesh,
        compiler_params=pltpu.CompilerParams(needs_layout_passes=False),
        scratch_types=[
            pltpu.VMEM((FPW * N,), jnp.float32),
            pltpu.VMEM((FPW * N,), jnp.float32),
            pltpu.VMEM((ECH,), jnp.int32),
            pltpu.VMEM((ECH,), jnp.int32),
            pltpu.VMEM((ECH,), jnp.int32),
            pltpu.VMEM((ECH,), jnp.int32),
            pltpu.SemaphoreType.DMA,
            pltpu.SemaphoreType.DMA,
            pltpu.SemaphoreType.DMA,
            pltpu.SemaphoreType.DMA,
        ],
    )


NEG = float("-inf")


def _probs_body(x_ref, W1_ref, b1_ref, W2_ref, b2_ref, W_root_ref,
                b_rel_ref, probs_ref, xroot_ref):
    x = x_ref[...]

    h = jnp.maximum(
        jnp.dot(x, W1_ref[...], preferred_element_type=jnp.float32)
        + b1_ref[...], 0.0)
    logits = (jnp.dot(h, W2_ref[...], preferred_element_type=jnp.float32)
              + b2_ref[...])
    m = jnp.max(logits, axis=-1, keepdims=True)
    e = jnp.exp(logits - m)
    probs_ref[...] = e / jnp.sum(e, axis=-1, keepdims=True)

    xroot_ref[...] = (
        jnp.dot(x, W_root_ref[...], preferred_element_type=jnp.float32)
        + b_rel_ref[...])


_probs = pl.pallas_call(
    _probs_body,
    out_shape=[
        jax.ShapeDtypeStruct((N, C), jnp.float32),
        jax.ShapeDtypeStruct((N, H), jnp.float32),
    ],
)


def _dense_body(aggT_ref, xroot_ref, batch_ref, batch_col_ref,
                W_rel_ref, W_out_ref, b_out_ref, ge_ref):
    aggT = aggT_ref[...]
    conv = (lax.dot_general(aggT, W_rel_ref[...],
                            (((0,), (0,)), ((), ())),
                            preferred_element_type=jnp.float32)
            + xroot_ref[...])

    bt_row = batch_ref[...]
    bt_col = batch_col_ref[...]
    seg = lax.broadcasted_iota(jnp.int32, (B, N), 0)
    onehot = (bt_row == seg).astype(jnp.float32)
    addp = jnp.dot(onehot, conv, preferred_element_type=jnp.float32)
    cnt = jnp.broadcast_to(jnp.sum(onehot, axis=1, keepdims=True), (B, H))
    meanp = addp / jnp.maximum(cnt, 1.0)

    rows = []
    for b in range(B):
        mask = bt_col == b
        rows.append(jnp.max(jnp.where(mask, conv, NEG), axis=0,
                            keepdims=True))
    maxp = jnp.concatenate(rows, axis=0)

    W_out = W_out_ref[...]
    ge_ref[...] = (
        jnp.dot(meanp, W_out[0:H, :], preferred_element_type=jnp.float32)
        + jnp.dot(maxp, W_out[H:2 * H, :], preferred_element_type=jnp.float32)
        + jnp.dot(addp, W_out[2 * H:3 * H, :],
                  preferred_element_type=jnp.float32)
        + b_out_ref[...])


_dense = pl.pallas_call(
    _dense_body,
    out_shape=jax.ShapeDtypeStruct((B, H), jnp.float32),
)


def kernel(x, edge_index, batch, W1, b1, W2, b2, W_rel, b_rel, W_root,
           W_out, b_out):
    xT = x.T.reshape(-1)
    aggT = _edge_agg()(xT, edge_index[0], edge_index[1]).reshape(D, N)
    probs, xroot = _probs(
        x, W1, b1.reshape(1, H), W2, b2.reshape(1, C), W_root,
        b_rel.reshape(1, H))
    ge = _dense(
        aggT, xroot, batch.reshape(1, N), batch.reshape(N, 1),
        W_rel, W_out, b_out.reshape(1, H))
    return (ge, probs)
